# baseline (speedup 1.0000x reference)
import functools

import jax
import jax.numpy as jnp
from jax import lax
from jax.experimental import pallas as pl
from jax.experimental.pallas import tpu as pltpu
from jax.experimental.pallas import tpu_sc as plsc

_NC = 2
_NS = 16
_NW = _NC * _NS
_NB = 2


def _proj_body(x_ref, wq_ref, bq_ref, wkv_ref, bkv_ref, s16_ref, blankk_ref,
               mask_ref, lg_ref, kv_ref, qmb_ref):
    x = x_ref[...]
    q = jnp.dot(x, wq_ref[...]) + bq_ref[...]
    kv_ref[...] = jnp.dot(x, wkv_ref[...]) + bkv_ref[...]
    maskadd = (1.0 - mask_ref[...]) * (-100.0) * lg_ref[0, 0]
    blank = jnp.clip(jnp.dot(q * blankk_ref[...], s16_ref[...]), -5.0, 5.0)
    qmb_ref[...] = jnp.concatenate([q, maskadd, blank], axis=1)


def _out_body(o_ref, wp_ref, bp_ref, out_ref):
    out_ref[...] = jnp.dot(o_ref[...], wp_ref[...]) + bp_ref[...]


def _make_sc_attn(BN, C, H, CH, M, per_w, tw):
    C2 = 2 * C
    PW = 8
    QW = 256
    G = tw // _NB
    mesh = plsc.VectorSubcoreMesh(core_axis_name="c", subcore_axis_name="s")

    @functools.partial(
        pl.kernel,
        mesh=mesh,
        out_type=jax.ShapeDtypeStruct((BN, C), jnp.float32),
        scratch_types=(
            [pltpu.VMEM((per_w,), jnp.int32),
             pltpu.VMEM((per_w,), jnp.int32),
             pltpu.VMEM((_NB * M, C2), jnp.float32),
             pltpu.VMEM((_NB * M, PW), jnp.float32),
             pltpu.VMEM((_NB, QW), jnp.float32),
             pltpu.VMEM((C,), jnp.float32),
             pltpu.VMEM((C, M), jnp.float32),
             pltpu.VMEM((_NB, C), jnp.float32)]
            + [pltpu.SemaphoreType.DMA] * (2 * _NB)
        ),
        compiler_params=pltpu.CompilerParams(use_tc_tiling_on_sc=False,
                                             needs_layout_passes=False),
    )
    def sc_attn(idx_hbm, pidx_hbm, kv_hbm, pre_hbm, qmb_hbm, bv_hbm, o_hbm,
                idxa, pidxa, kvring, pering, qring, bvbuf, kt, outbuf,
                *sems):
        semg = sems[:_NB]
        semo = sems[_NB:]
        iota = lax.iota(jnp.int32, 16)
        _dnums = lax.GatherDimensionNumbers(
            offset_dims=(), collapsed_slice_dims=(0,), start_index_map=(0,))

        def splat(vec, lane):
            idx = jnp.full((16, 1), lane, jnp.int32)
            return lax.gather(vec, idx, _dnums, (1,),
                              mode=lax.GatherScatterMode.PROMISE_IN_BOUNDS)

        def sum_splat(vec):
            return splat(jnp.cumsum(vec), 15)

        def max_splat(vec):
            return splat(plsc.cummax(vec), 15)
        wid = lax.axis_index("s") * _NC + lax.axis_index("c")
        t0 = wid * tw
        w0 = wid * per_w
        pltpu.sync_copy(idx_hbm.at[pl.ds(w0, per_w)], idxa)
        pltpu.sync_copy(pidx_hbm.at[pl.ds(w0, per_w)], pidxa)
        pltpu.sync_copy(bv_hbm, bvbuf)

        def in_descs(tl, b):
            return (
                pltpu.make_async_copy(
                    kv_hbm.at[idxa.at[pl.ds(tl * M, M)]],
                    kvring.at[pl.ds(b * M, M)], semg[b]),
                pltpu.make_async_copy(
                    pre_hbm.at[pidxa.at[pl.ds(tl * M, M)]],
                    pering.at[pl.ds(b * M, M)], semg[b]),
                pltpu.make_async_copy(qmb_hbm.at[t0 + tl], qring.at[b],
                                      semg[b]),
            )

        def out_desc(tl, b):
            return pltpu.make_async_copy(outbuf.at[b], o_hbm.at[t0 + tl],
                                         semo[b])

        for b in range(_NB):
            for d in in_descs(b, b):
                d.start()

        def body(g, carry):
            for b in range(_NB):
                tl = g * _NB + b
                for d in in_descs(tl, b):
                    d.wait()

                @pl.when(g >= 1)
                def _():
                    out_desc(tl, b).wait()

                qv = [qring[b, pl.ds(h * CH + o, 16)]
                      for h in range(H) for o in (0, 16)]
                bvv = [bvbuf[pl.ds(h * CH + o, 16)]
                       for h in range(H) for o in (0, 16)]
                maskv = [qring[b, pl.ds(C + j * 16, 16)] for j in range(3)]

                z16 = jnp.zeros((16,), jnp.float32)
                for m in range(M):
                    r = b * M + m
                    mful = jnp.full((16,), m, jnp.int32)
                    for h in range(H):
                        for o in (0, 16):
                            plsc.store_scatter(
                                kt, [h * CH + o + iota, mful],
                                kvring[r, pl.ds(h * 2 * CH + o, 16)])
                svec = []
                for h in range(H):
                    qs = [splat(qv[2 * h + (c // 16)], c % 16)
                          for c in range(CH)]
                    row = []
                    for j in range(3):
                        acc = z16
                        for c in range(CH):
                            acc = acc + kt[h * CH + c, pl.ds(j * 16, 16)] \
                                * qs[c]
                        row.append(acc)
                    svec.append(row)

                blv = qring[b, pl.ds(C + M, 16)]
                avec = []
                for h in range(H):
                    bl = splat(blv, h)
                    sv = []
                    for j in range(3):
                        pe = plsc.load_gather(
                            pering,
                            [iota + (b * M + j * 16), jnp.full((16,), h,
                                                               jnp.int32)])
                        sv.append(svec[h][j] + pe + maskv[j])
                    sv3 = jnp.where(iota == 0, bl, -1e30)
                    mx = jnp.maximum(
                        max_splat(jnp.maximum(jnp.maximum(sv[0], sv[1]),
                                              sv[2])),
                        bl)
                    e = [jnp.exp(x - mx) for x in (sv[0], sv[1], sv[2], sv3)]
                    denv = sum_splat(e[0] + e[1] + e[2] + e[3])
                    avec.append([x / denv for x in e])

                acc = []
                for h in range(H):
                    a48 = splat(avec[h][3], 0)
                    acc.append(bvv[2 * h] * a48)
                    acc.append(bvv[2 * h + 1] * a48)
                for m in range(M):
                    r = b * M + m
                    j, l = divmod(m, 16)
                    for h in range(H):
                        a = splat(avec[h][j], l)
                        acc[2 * h] += kvring[r, pl.ds(h * 2 * CH + CH, 16)] * a
                        acc[2 * h + 1] += kvring[
                            r, pl.ds(h * 2 * CH + CH + 16, 16)] * a
                for h in range(H):
                    outbuf[b, pl.ds(h * CH, 16)] = acc[2 * h]
                    outbuf[b, pl.ds(h * CH + 16, 16)] = acc[2 * h + 1]
                out_desc(tl, b).start()

                @pl.when(g < G - 1)
                def _():
                    for d in in_descs(tl + _NB, b):
                        d.start()

            return carry

        lax.fori_loop(0, G, body, 0)
        for b in range(_NB):
            out_desc((G - 1) * _NB + b, b).wait()

    return sc_attn


def kernel(feat, member_idx, cluster_mask, pe_idx, global_attn, pre_table,
           W_q, b_q, W_kv, b_kv, blank_k, blank_v, W_pe, b_pe, W_proj, b_proj):
    B, N, C = feat.shape
    M = member_idx.shape[-1]
    H = W_pe.shape[1]
    CH = C // H
    C2 = 2 * C
    T = pre_table.shape[0]
    BN = B * N
    R = BN * M
    scale = jnp.float32(CH) ** -0.5

    f32 = jnp.float32
    x = feat.reshape(BN, C)
    Wq_s = W_q * scale
    bq_s = (b_q * scale).reshape(1, C)
    S16 = ((jnp.arange(C)[:, None] // CH) == jnp.arange(16)[None, :]
           ).astype(f32)
    lg = jnp.logical_not(global_attn).astype(f32).reshape(1, 1)
    mask2 = cluster_mask.reshape(BN, M)

    TB1 = 256
    g1 = BN // TB1
    kv2, qmb = pl.pallas_call(
        _proj_body,
        grid=(g1,),
        in_specs=[
            pl.BlockSpec((TB1, C), lambda i: (i, 0)),
            pl.BlockSpec((C, C), lambda i: (0, 0)),
            pl.BlockSpec((1, C), lambda i: (0, 0)),
            pl.BlockSpec((C, C2), lambda i: (0, 0)),
            pl.BlockSpec((1, C2), lambda i: (0, 0)),
            pl.BlockSpec((C, 16), lambda i: (0, 0)),
            pl.BlockSpec((1, C), lambda i: (0, 0)),
            pl.BlockSpec((TB1, M), lambda i: (i, 0)),
            pl.BlockSpec((1, 1), lambda i: (0, 0), memory_space=pltpu.SMEM),
        ],
        out_specs=[
            pl.BlockSpec((TB1, C2), lambda i: (i, 0)),
            pl.BlockSpec((TB1, 256), lambda i: (i, 0)),
        ],
        out_shape=[jax.ShapeDtypeStruct((BN, C2), f32),
                   jax.ShapeDtypeStruct((BN, 256), f32)],
    )(x, Wq_s, bq_s, W_kv, b_kv.reshape(1, C2), S16, blank_k.reshape(1, C),
      mask2, lg)

    gidx = (member_idx.astype(jnp.int32)
            + (jnp.arange(B, dtype=jnp.int32) * N)[:, None, None]).reshape(R)
    pidx = pe_idx.astype(jnp.int32).reshape(R)
    TP = (T + 7) // 8 * 8
    pre8 = jnp.zeros((TP, 8), f32).at[:T, :5].set(pre_table)
    Wpe8 = jnp.zeros((8, 8), f32).at[:5, :H].set(W_pe)
    bpe8 = jnp.zeros((1, 8), f32).at[0, :H].set(b_pe)
    pe8 = pl.pallas_call(
        _out_body,
        grid=(1,),
        in_specs=[
            pl.BlockSpec((TP, 8), lambda i: (0, 0)),
            pl.BlockSpec((8, 8), lambda i: (0, 0)),
            pl.BlockSpec((1, 8), lambda i: (0, 0)),
        ],
        out_specs=pl.BlockSpec((TP, 8), lambda i: (0, 0)),
        out_shape=jax.ShapeDtypeStruct((TP, 8), f32),
    )(pre8, Wpe8, bpe8)

    per_w = R // _NW
    tw = BN // _NW
    o = _make_sc_attn(BN, C, H, CH, M, per_w, tw)(
        gidx, pidx, kv2, pe8, qmb, blank_v)

    TB3 = 256
    out = pl.pallas_call(
        _out_body,
        grid=(BN // TB3,),
        in_specs=[
            pl.BlockSpec((TB3, C), lambda i: (i, 0)),
            pl.BlockSpec((C, C), lambda i: (0, 0)),
            pl.BlockSpec((1, C), lambda i: (0, 0)),
        ],
        out_specs=pl.BlockSpec((TB3, C), lambda i: (i, 0)),
        out_shape=jax.ShapeDtypeStruct((BN, C), f32),
    )(o, W_proj, b_proj.reshape(1, C))

    return out.reshape(B, N, C)


# speedup vs baseline: 1.8353x; 1.8353x over previous
import functools

import jax
import jax.numpy as jnp
from jax import lax
from jax.experimental import pallas as pl
from jax.experimental.pallas import tpu as pltpu
from jax.experimental.pallas import tpu_sc as plsc

_NC = 2
_NS = 16
_NW = _NC * _NS
_NB = 2


def _proj_body(x_ref, wq_ref, bq_ref, wkv_ref, bkv_ref, s16_ref, blankk_ref,
               mask_ref, lg_ref, kv_ref, qmb_ref):
    x = x_ref[...]
    q = jnp.dot(x, wq_ref[...]) + bq_ref[...]
    kv_ref[...] = jnp.dot(x, wkv_ref[...]) + bkv_ref[...]
    maskadd = (1.0 - mask_ref[...]) * (-100.0) * lg_ref[0, 0]
    blank = jnp.clip(jnp.dot(q * blankk_ref[...], s16_ref[...]), -5.0, 5.0)
    qmb_ref[...] = jnp.concatenate([q, maskadd, blank], axis=1)


def _out_body(o_ref, wp_ref, bp_ref, out_ref):
    out_ref[...] = jnp.dot(o_ref[...], wp_ref[...]) + bp_ref[...]


def _make_sc_attn(BN, C, H, CH, M, per_w, tw):
    C2 = 2 * C
    PW = 8
    QW = 256
    G = tw // _NB
    mesh = plsc.VectorSubcoreMesh(core_axis_name="c", subcore_axis_name="s")

    @functools.partial(
        pl.kernel,
        mesh=mesh,
        out_type=jax.ShapeDtypeStruct((BN, C), jnp.float32),
        scratch_types=(
            [pltpu.VMEM((per_w,), jnp.int32),
             pltpu.VMEM((per_w,), jnp.int32),
             pltpu.VMEM((_NB * M, C2), jnp.float32),
             pltpu.VMEM((_NB * M, PW), jnp.float32),
             pltpu.VMEM((_NB, QW), jnp.float32),
             pltpu.VMEM((C,), jnp.float32),
             pltpu.VMEM((_NB, C), jnp.float32)]
            + [pltpu.SemaphoreType.DMA] * (2 * _NB)
        ),
        compiler_params=pltpu.CompilerParams(use_tc_tiling_on_sc=False,
                                             needs_layout_passes=False),
    )
    def sc_attn(idx_hbm, pidx_hbm, kv_hbm, pre_hbm, qmb_hbm, bv_hbm, o_hbm,
                idxa, pidxa, kvring, pering, qring, bvbuf, outbuf,
                *sems):
        semg = sems[:_NB]
        semo = sems[_NB:]
        iota = lax.iota(jnp.int32, 16)
        _dnums = lax.GatherDimensionNumbers(
            offset_dims=(), collapsed_slice_dims=(0,), start_index_map=(0,))

        def perm(vec, idx):
            return lax.gather(vec, idx[:, None], _dnums, (1,),
                              mode=lax.GatherScatterMode.PROMISE_IN_BOUNDS)

        def splat(vec, lane):
            return perm(vec, jnp.full((16,), lane, jnp.int32))

        bfly = [iota ^ (1 << k) for k in range(4)]

        def sum_splat(vec):
            for ix in bfly:
                vec = vec + perm(vec, ix)
            return vec

        def max_splat(vec):
            return splat(plsc.cummax(vec), 15)
        wid = lax.axis_index("s") * _NC + lax.axis_index("c")
        t0 = wid * tw
        w0 = wid * per_w
        pltpu.sync_copy(idx_hbm.at[pl.ds(w0, per_w)], idxa)
        pltpu.sync_copy(pidx_hbm.at[pl.ds(w0, per_w)], pidxa)
        pltpu.sync_copy(bv_hbm, bvbuf)

        def in_descs(tl, b):
            return (
                pltpu.make_async_copy(
                    kv_hbm.at[idxa.at[pl.ds(tl * M, M)]],
                    kvring.at[pl.ds(b * M, M)], semg[b]),
                pltpu.make_async_copy(
                    pre_hbm.at[pidxa.at[pl.ds(tl * M, M)]],
                    pering.at[pl.ds(b * M, M)], semg[b]),
                pltpu.make_async_copy(qmb_hbm.at[t0 + tl], qring.at[b],
                                      semg[b]),
            )

        def out_desc(tl, b):
            return pltpu.make_async_copy(outbuf.at[b], o_hbm.at[t0 + tl],
                                         semo[b])

        for b in range(_NB):
            for d in in_descs(b, b):
                d.start()

        def body(g, carry):
            for b in range(_NB):
                tl = g * _NB + b
                for d in in_descs(tl, b):
                    d.wait()

                @pl.when(g >= 1)
                def _():
                    out_desc(tl, b).wait()

                qv = [qring[b, pl.ds(h * CH + o, 16)]
                      for h in range(H) for o in (0, 16)]
                bvv = [bvbuf[pl.ds(h * CH + o, 16)]
                       for h in range(H) for o in (0, 16)]
                maskv = [qring[b, pl.ds(C + j * 16, 16)] for j in range(3)]

                z16 = jnp.zeros((16,), jnp.float32)
                svec = [[z16, z16, z16] for _ in range(H)]
                for m in range(M):
                    r = b * M + m
                    j, l = divmod(m, 16)
                    for h in range(H):
                        t = (kvring[r, pl.ds(h * 2 * CH, 16)] * qv[2 * h]
                             + kvring[r, pl.ds(h * 2 * CH + 16, 16)]
                             * qv[2 * h + 1])
                        svec[h][j] = jnp.where(iota == l, sum_splat(t),
                                               svec[h][j])

                blv = qring[b, pl.ds(C + M, 16)]
                avec = []
                for h in range(H):
                    bl = splat(blv, h)
                    sv = []
                    for j in range(3):
                        pe = plsc.load_gather(
                            pering,
                            [iota + (b * M + j * 16), jnp.full((16,), h,
                                                               jnp.int32)])
                        sv.append(svec[h][j] + pe + maskv[j])
                    sv3 = jnp.where(iota == 0, bl, -1e30)
                    mx = jnp.maximum(
                        max_splat(jnp.maximum(jnp.maximum(sv[0], sv[1]),
                                              sv[2])),
                        bl)
                    e = [jnp.exp(x - mx) for x in (sv[0], sv[1], sv[2], sv3)]
                    denv = sum_splat(e[0] + e[1] + e[2] + e[3])
                    avec.append([x / denv for x in e])

                acc = []
                for h in range(H):
                    a48 = splat(avec[h][3], 0)
                    acc.append(bvv[2 * h] * a48)
                    acc.append(bvv[2 * h + 1] * a48)
                for m in range(M):
                    r = b * M + m
                    j, l = divmod(m, 16)
                    for h in range(H):
                        a = splat(avec[h][j], l)
                        acc[2 * h] += kvring[r, pl.ds(h * 2 * CH + CH, 16)] * a
                        acc[2 * h + 1] += kvring[
                            r, pl.ds(h * 2 * CH + CH + 16, 16)] * a
                for h in range(H):
                    outbuf[b, pl.ds(h * CH, 16)] = acc[2 * h]
                    outbuf[b, pl.ds(h * CH + 16, 16)] = acc[2 * h + 1]
                out_desc(tl, b).start()

                @pl.when(g < G - 1)
                def _():
                    for d in in_descs(tl + _NB, b):
                        d.start()

            return carry

        lax.fori_loop(0, G, body, 0)
        for b in range(_NB):
            out_desc((G - 1) * _NB + b, b).wait()

    return sc_attn


def kernel(feat, member_idx, cluster_mask, pe_idx, global_attn, pre_table,
           W_q, b_q, W_kv, b_kv, blank_k, blank_v, W_pe, b_pe, W_proj, b_proj):
    B, N, C = feat.shape
    M = member_idx.shape[-1]
    H = W_pe.shape[1]
    CH = C // H
    C2 = 2 * C
    T = pre_table.shape[0]
    BN = B * N
    R = BN * M
    scale = jnp.float32(CH) ** -0.5

    f32 = jnp.float32
    x = feat.reshape(BN, C)
    Wq_s = W_q * scale
    bq_s = (b_q * scale).reshape(1, C)
    S16 = ((jnp.arange(C)[:, None] // CH) == jnp.arange(16)[None, :]
           ).astype(f32)
    lg = jnp.logical_not(global_attn).astype(f32).reshape(1, 1)
    mask2 = cluster_mask.reshape(BN, M)

    TB1 = 256
    g1 = BN // TB1
    kv2, qmb = pl.pallas_call(
        _proj_body,
        grid=(g1,),
        in_specs=[
            pl.BlockSpec((TB1, C), lambda i: (i, 0)),
            pl.BlockSpec((C, C), lambda i: (0, 0)),
            pl.BlockSpec((1, C), lambda i: (0, 0)),
            pl.BlockSpec((C, C2), lambda i: (0, 0)),
            pl.BlockSpec((1, C2), lambda i: (0, 0)),
            pl.BlockSpec((C, 16), lambda i: (0, 0)),
            pl.BlockSpec((1, C), lambda i: (0, 0)),
            pl.BlockSpec((TB1, M), lambda i: (i, 0)),
            pl.BlockSpec((1, 1), lambda i: (0, 0), memory_space=pltpu.SMEM),
        ],
        out_specs=[
            pl.BlockSpec((TB1, C2), lambda i: (i, 0)),
            pl.BlockSpec((TB1, 256), lambda i: (i, 0)),
        ],
        out_shape=[jax.ShapeDtypeStruct((BN, C2), f32),
                   jax.ShapeDtypeStruct((BN, 256), f32)],
    )(x, Wq_s, bq_s, W_kv, b_kv.reshape(1, C2), S16, blank_k.reshape(1, C),
      mask2, lg)

    gidx = (member_idx.astype(jnp.int32)
            + (jnp.arange(B, dtype=jnp.int32) * N)[:, None, None]).reshape(R)
    pidx = pe_idx.astype(jnp.int32).reshape(R)
    TP = (T + 7) // 8 * 8
    pre8 = jnp.zeros((TP, 8), f32).at[:T, :5].set(pre_table)
    Wpe8 = jnp.zeros((8, 8), f32).at[:5, :H].set(W_pe)
    bpe8 = jnp.zeros((1, 8), f32).at[0, :H].set(b_pe)
    pe8 = pl.pallas_call(
        _out_body,
        grid=(1,),
        in_specs=[
            pl.BlockSpec((TP, 8), lambda i: (0, 0)),
            pl.BlockSpec((8, 8), lambda i: (0, 0)),
            pl.BlockSpec((1, 8), lambda i: (0, 0)),
        ],
        out_specs=pl.BlockSpec((TP, 8), lambda i: (0, 0)),
        out_shape=jax.ShapeDtypeStruct((TP, 8), f32),
    )(pre8, Wpe8, bpe8)

    per_w = R // _NW
    tw = BN // _NW
    o = _make_sc_attn(BN, C, H, CH, M, per_w, tw)(
        gidx, pidx, kv2, pe8, qmb, blank_v)

    TB3 = 256
    out = pl.pallas_call(
        _out_body,
        grid=(BN // TB3,),
        in_specs=[
            pl.BlockSpec((TB3, C), lambda i: (i, 0)),
            pl.BlockSpec((C, C), lambda i: (0, 0)),
            pl.BlockSpec((1, C), lambda i: (0, 0)),
        ],
        out_specs=pl.BlockSpec((TB3, C), lambda i: (i, 0)),
        out_shape=jax.ShapeDtypeStruct((BN, C), f32),
    )(o, W_proj, b_proj.reshape(1, C))

    return out.reshape(B, N, C)


# speedup vs baseline: 2.5886x; 1.4104x over previous
import functools

import jax
import jax.numpy as jnp
from jax import lax
from jax.experimental import pallas as pl
from jax.experimental.pallas import tpu as pltpu
from jax.experimental.pallas import tpu_sc as plsc

_NC = 2
_NS = 16
_NW = _NC * _NS


def _proj_body(x_ref, wq_ref, bq_ref, wk_ref, bk_ref, wv_ref, bv_ref,
               q_ref, kvp_ref):
    x = x_ref[...]
    tb = x.shape[0]
    q_ref[...] = jnp.dot(x, wq_ref[...]) + bq_ref[...]
    k = jnp.dot(x, wk_ref[...]) + bk_ref[...]
    v = jnp.dot(x, wv_ref[...]) + bv_ref[...]
    ku = lax.bitcast_convert_type(k.astype(jnp.bfloat16),
                                  jnp.uint16).astype(jnp.uint32)
    vu = lax.bitcast_convert_type(v.astype(jnp.bfloat16),
                                  jnp.uint16).astype(jnp.uint32)
    w = lax.bitcast_convert_type((ku << 16) | vu, jnp.float32)
    kvp_ref[...] = jnp.concatenate(
        [w, jnp.zeros((tb, 64), jnp.float32)], axis=1)


def _attn_body(q_ref, kvg_ref, peg_ref, mask_ref, lg_ref, s_ref, srep_ref,
               wpe_ref, bpe_ref, blankk_ref, blankv_ref, wproj_ref, bproj_ref,
               out_ref, *, tb, m):
    c = q_ref.shape[-1]
    q = q_ref[...]
    s_sel = s_ref[...]
    wu = lax.bitcast_convert_type(kvg_ref[...][:, :c], jnp.uint32)
    kg = lax.bitcast_convert_type(wu & jnp.uint32(0xFFFF0000), jnp.float32)
    vg = lax.bitcast_convert_type(wu << 16, jnp.float32)
    qe = jnp.broadcast_to(q[:, None, :], (tb, m, c)).reshape(tb * m, c)
    scores = jnp.dot(qe * kg, s_sel)
    pe = jnp.dot(peg_ref[...], wpe_ref[...]) + bpe_ref[...]
    lg = lg_ref[0, 0]
    s3 = scores.reshape(tb, m, 8) + pe.reshape(tb, m, 8)
    s3 = s3 + ((1.0 - mask_ref[...]) * (-100.0) * lg)[:, :, None]
    bl = jnp.clip(jnp.dot(q * blankk_ref[...], s_sel), -5.0, 5.0)
    mx = jnp.maximum(jnp.max(s3, axis=1), bl)
    e3 = jnp.exp(s3 - mx[:, None, :])
    eb = jnp.exp(bl - mx)
    den = jnp.sum(e3, axis=1) + eb
    attn = (e3 / den[:, None, :]).reshape(tb * m, 8)
    ar = jnp.dot(attn, srep_ref[...])
    out = jnp.sum((ar * vg).reshape(tb, m, c), axis=1)
    out = out + jnp.dot(eb / den, srep_ref[...]) * blankv_ref[...]
    out_ref[...] = jnp.dot(out, wproj_ref[...]) + bproj_ref[...]


def _make_sc_gather(rows, width, chk, per_w, nbuf, tc_tiling):
    n_chunks = per_w // chk
    t_steps = n_chunks // nbuf
    mesh = plsc.VectorSubcoreMesh(core_axis_name="c", subcore_axis_name="s")

    @functools.partial(
        pl.kernel,
        mesh=mesh,
        out_type=jax.ShapeDtypeStruct((rows, width), jnp.float32),
        scratch_types=(
            [pltpu.VMEM((per_w,), jnp.int32),
             pltpu.VMEM((nbuf * chk, width), jnp.float32)]
            + [pltpu.SemaphoreType.DMA] * (2 * nbuf)
        ),
        compiler_params=pltpu.CompilerParams(use_tc_tiling_on_sc=tc_tiling),
    )
    def sc_gather(idx_hbm, table_hbm, out_hbm, idx_all, bufs, *sems):
        semg = sems[:nbuf]
        semw = sems[nbuf:]
        wid = lax.axis_index("s") * _NC + lax.axis_index("c")
        w0 = wid * per_w
        pltpu.sync_copy(idx_hbm.at[pl.ds(w0, per_w)], idx_all)

        def gat(i, b):
            return pltpu.make_async_copy(
                table_hbm.at[idx_all.at[pl.ds(i * chk, chk)]],
                bufs.at[pl.ds(b * chk, chk)], semg[b])

        def sto(i, b):
            return pltpu.make_async_copy(
                bufs.at[pl.ds(b * chk, chk)],
                out_hbm.at[pl.ds(w0 + i * chk, chk)], semw[b])

        for b in range(nbuf):
            gat(b, b).start()

        def body(t, carry):
            for b in range(nbuf):
                i = t * nbuf + b
                gat(i, b).wait()
                sto(i, b).start()

                @pl.when(t < t_steps - 1)
                def _():
                    sto(i, b).wait()
                    gat(i + nbuf, b).start()

            return carry

        lax.fori_loop(0, t_steps, body, 0)
        for b in range(nbuf):
            sto((t_steps - 1) * nbuf + b, b).wait()

    return sc_gather


def kernel(feat, member_idx, cluster_mask, pe_idx, global_attn, pre_table,
           W_q, b_q, W_kv, b_kv, blank_k, blank_v, W_pe, b_pe, W_proj, b_proj):
    B, N, C = feat.shape
    M = member_idx.shape[-1]
    H = W_pe.shape[1]
    CH = C // H
    T = pre_table.shape[0]
    BN = B * N
    R = BN * M
    CP = 256
    scale = jnp.float32(CH) ** -0.5

    f32 = jnp.float32
    x = feat.reshape(BN, C)

    hcol = jnp.arange(C)
    kcols = (hcol // CH) * (2 * CH) + (hcol % CH)
    vcols = kcols + CH
    Wk = W_kv[:, kcols]
    Wv = W_kv[:, vcols]
    bk = b_kv[kcols].reshape(1, C)
    bv = b_kv[vcols].reshape(1, C)

    TB1 = 256
    g1 = BN // TB1
    q2, kvp = pl.pallas_call(
        _proj_body,
        grid=(g1,),
        in_specs=[
            pl.BlockSpec((TB1, C), lambda i: (i, 0)),
            pl.BlockSpec((C, C), lambda i: (0, 0)),
            pl.BlockSpec((1, C), lambda i: (0, 0)),
            pl.BlockSpec((C, C), lambda i: (0, 0)),
            pl.BlockSpec((1, C), lambda i: (0, 0)),
            pl.BlockSpec((C, C), lambda i: (0, 0)),
            pl.BlockSpec((1, C), lambda i: (0, 0)),
        ],
        out_specs=[
            pl.BlockSpec((TB1, C), lambda i: (i, 0)),
            pl.BlockSpec((TB1, CP), lambda i: (i, 0)),
        ],
        out_shape=[jax.ShapeDtypeStruct((BN, C), f32),
                   jax.ShapeDtypeStruct((BN, CP), f32)],
    )(x, W_q * scale, (b_q * scale).reshape(1, C), Wk, bk, Wv, bv)

    gidx = (member_idx.astype(jnp.int32)
            + (jnp.arange(B, dtype=jnp.int32) * N)[:, None, None]).reshape(R)
    pidx = pe_idx.astype(jnp.int32).reshape(R)
    PW = 8
    pre8 = jnp.zeros((T, PW), f32).at[:, :5].set(pre_table)

    per_w = R // _NW
    kvg = _make_sc_gather(R, CP, 64, per_w, 3, True)(gidx, kvp)
    peg = _make_sc_gather(R, PW, 128, per_w, 3, False)(pidx, pre8)

    S = ((hcol[:, None] // CH) == jnp.arange(8)[None, :]).astype(f32)
    Srep = S.T
    Wpe8 = jnp.zeros((PW, 8), f32).at[:5, :H].set(W_pe)
    bpe8 = jnp.zeros((1, 8), f32).at[0, :H].set(b_pe)
    lg = jnp.logical_not(global_attn).astype(f32).reshape(1, 1)
    mask2 = cluster_mask.reshape(BN, M)

    TB2 = 64
    g2 = BN // TB2
    out = pl.pallas_call(
        functools.partial(_attn_body, tb=TB2, m=M),
        grid=(g2,),
        in_specs=[
            pl.BlockSpec((TB2, C), lambda i: (i, 0)),
            pl.BlockSpec((TB2 * M, CP), lambda i: (i, 0)),
            pl.BlockSpec((TB2 * M, PW), lambda i: (i, 0)),
            pl.BlockSpec((TB2, M), lambda i: (i, 0)),
            pl.BlockSpec((1, 1), lambda i: (0, 0), memory_space=pltpu.SMEM),
            pl.BlockSpec((C, 8), lambda i: (0, 0)),
            pl.BlockSpec((8, C), lambda i: (0, 0)),
            pl.BlockSpec((PW, 8), lambda i: (0, 0)),
            pl.BlockSpec((1, 8), lambda i: (0, 0)),
            pl.BlockSpec((1, C), lambda i: (0, 0)),
            pl.BlockSpec((1, C), lambda i: (0, 0)),
            pl.BlockSpec((C, C), lambda i: (0, 0)),
            pl.BlockSpec((1, C), lambda i: (0, 0)),
        ],
        out_specs=pl.BlockSpec((TB2, C), lambda i: (i, 0)),
        out_shape=jax.ShapeDtypeStruct((BN, C), f32),
    )(q2, kvg, peg, mask2, lg, S, Srep, Wpe8, bpe8,
      blank_k.reshape(1, C), blank_v.reshape(1, C), W_proj,
      b_proj.reshape(1, C))

    return out.reshape(B, N, C)


# speedup vs baseline: 2.6954x; 1.0413x over previous
import functools

import jax
import jax.numpy as jnp
from jax import lax
from jax.experimental import pallas as pl
from jax.experimental.pallas import tpu as pltpu
from jax.experimental.pallas import tpu_sc as plsc

_NC = 2
_NS = 16
_NW = _NC * _NS


def _proj_body(x_ref, wq_ref, bq_ref, wk_ref, bk_ref, wv_ref, bv_ref,
               q_ref, kvp_ref):
    x = x_ref[...]
    tb = x.shape[0]
    q_ref[...] = jnp.dot(x, wq_ref[...]) + bq_ref[...]
    k = jnp.dot(x, wk_ref[...]) + bk_ref[...]
    v = jnp.dot(x, wv_ref[...]) + bv_ref[...]
    ku = lax.bitcast_convert_type(k.astype(jnp.bfloat16),
                                  jnp.uint16).astype(jnp.uint32)
    vu = lax.bitcast_convert_type(v.astype(jnp.bfloat16),
                                  jnp.uint16).astype(jnp.uint32)
    w = lax.bitcast_convert_type((ku << 16) | vu, jnp.float32)
    kvp_ref[...] = jnp.concatenate(
        [w, jnp.zeros((tb, 64), jnp.float32)], axis=1)


def _attn_body(q_ref, kvg_ref, peg_ref, mask_ref, lg_ref, s_ref, srep_ref,
               wpe_ref, bpe_ref, blankk_ref, blankv_ref, wproj_ref, bproj_ref,
               out_ref, *, tb, m):
    c = q_ref.shape[-1]
    q = q_ref[...]
    s_sel = s_ref[...]
    wu = lax.bitcast_convert_type(kvg_ref[...][:, :c], jnp.uint32)
    kg = lax.bitcast_convert_type(wu & jnp.uint32(0xFFFF0000), jnp.float32)
    vg = lax.bitcast_convert_type(wu << 16, jnp.float32)
    qe = jnp.broadcast_to(q[:, None, :], (tb, m, c)).reshape(tb * m, c)
    scores = jnp.dot(qe * kg, s_sel)
    pe = jnp.dot(peg_ref[...], wpe_ref[...]) + bpe_ref[...]
    lg = lg_ref[0, 0]
    s3 = scores.reshape(tb, m, 8) + pe.reshape(tb, m, 8)
    s3 = s3 + ((1.0 - mask_ref[...]) * (-100.0) * lg)[:, :, None]
    bl = jnp.clip(jnp.dot(q * blankk_ref[...], s_sel), -5.0, 5.0)
    mx = jnp.maximum(jnp.max(s3, axis=1), bl)
    e3 = jnp.exp(s3 - mx[:, None, :])
    eb = jnp.exp(bl - mx)
    den = jnp.sum(e3, axis=1) + eb
    attn = (e3 / den[:, None, :]).reshape(tb * m, 8)
    ar = jnp.dot(attn, srep_ref[...])
    out = jnp.sum((ar * vg).reshape(tb, m, c), axis=1)
    out = out + jnp.dot(eb / den, srep_ref[...]) * blankv_ref[...]
    out_ref[...] = jnp.dot(out, wproj_ref[...]) + bproj_ref[...]


def _make_sc_gather(rows, width, chk, per_w, nbuf, tc_tiling):
    n_chunks = per_w // chk
    t_steps = n_chunks // nbuf
    mesh = plsc.VectorSubcoreMesh(core_axis_name="c", subcore_axis_name="s")

    @functools.partial(
        pl.kernel,
        mesh=mesh,
        out_type=jax.ShapeDtypeStruct((rows, width), jnp.float32),
        scratch_types=(
            [pltpu.VMEM((per_w,), jnp.int32),
             pltpu.VMEM((nbuf * chk, width), jnp.float32)]
            + [pltpu.SemaphoreType.DMA] * (2 * nbuf)
        ),
        compiler_params=pltpu.CompilerParams(use_tc_tiling_on_sc=tc_tiling),
    )
    def sc_gather(idx_hbm, table_hbm, out_hbm, idx_all, bufs, *sems):
        semg = sems[:nbuf]
        semw = sems[nbuf:]
        wid = lax.axis_index("s") * _NC + lax.axis_index("c")
        w0 = wid * per_w
        pltpu.sync_copy(idx_hbm.at[pl.ds(w0, per_w)], idx_all)

        def gat(i, b):
            return pltpu.make_async_copy(
                table_hbm.at[idx_all.at[pl.ds(i * chk, chk)]],
                bufs.at[pl.ds(b * chk, chk)], semg[b])

        def sto(i, b):
            return pltpu.make_async_copy(
                bufs.at[pl.ds(b * chk, chk)],
                out_hbm.at[pl.ds(w0 + i * chk, chk)], semw[b])

        for b in range(nbuf):
            gat(b, b).start()

        def body(t, carry):
            for b in range(nbuf):
                i = t * nbuf + b
                gat(i, b).wait()
                sto(i, b).start()

                @pl.when(t < t_steps - 1)
                def _():
                    sto(i, b).wait()
                    gat(i + nbuf, b).start()

            return carry

        lax.fori_loop(0, t_steps, body, 0)
        for b in range(nbuf):
            sto((t_steps - 1) * nbuf + b, b).wait()

    return sc_gather


def kernel(feat, member_idx, cluster_mask, pe_idx, global_attn, pre_table,
           W_q, b_q, W_kv, b_kv, blank_k, blank_v, W_pe, b_pe, W_proj, b_proj):
    B, N, C = feat.shape
    M = member_idx.shape[-1]
    H = W_pe.shape[1]
    CH = C // H
    T = pre_table.shape[0]
    BN = B * N
    R = BN * M
    CP = 256
    scale = jnp.float32(CH) ** -0.5

    f32 = jnp.float32
    x = feat.reshape(BN, C)

    hcol = jnp.arange(C)
    kcols = (hcol // CH) * (2 * CH) + (hcol % CH)
    vcols = kcols + CH
    Wk = W_kv[:, kcols]
    Wv = W_kv[:, vcols]
    bk = b_kv[kcols].reshape(1, C)
    bv = b_kv[vcols].reshape(1, C)

    TB1 = 256
    g1 = BN // TB1
    q2, kvp = pl.pallas_call(
        _proj_body,
        grid=(g1,),
        in_specs=[
            pl.BlockSpec((TB1, C), lambda i: (i, 0)),
            pl.BlockSpec((C, C), lambda i: (0, 0)),
            pl.BlockSpec((1, C), lambda i: (0, 0)),
            pl.BlockSpec((C, C), lambda i: (0, 0)),
            pl.BlockSpec((1, C), lambda i: (0, 0)),
            pl.BlockSpec((C, C), lambda i: (0, 0)),
            pl.BlockSpec((1, C), lambda i: (0, 0)),
        ],
        out_specs=[
            pl.BlockSpec((TB1, C), lambda i: (i, 0)),
            pl.BlockSpec((TB1, CP), lambda i: (i, 0)),
        ],
        out_shape=[jax.ShapeDtypeStruct((BN, C), f32),
                   jax.ShapeDtypeStruct((BN, CP), f32)],
    )(x, W_q * scale, (b_q * scale).reshape(1, C), Wk, bk, Wv, bv)

    gidx = (member_idx.astype(jnp.int32)
            + (jnp.arange(B, dtype=jnp.int32) * N)[:, None, None]).reshape(R)
    pidx = pe_idx.astype(jnp.int32).reshape(R)
    PW = 8
    pre8 = jnp.zeros((T, PW), f32).at[:, :5].set(pre_table)

    per_w = R // _NW
    kvg = _make_sc_gather(R, CP, 128, per_w, 3, True)(gidx, kvp)
    peg = _make_sc_gather(R, PW, 128, per_w, 3, False)(pidx, pre8)

    S = ((hcol[:, None] // CH) == jnp.arange(8)[None, :]).astype(f32)
    Srep = S.T
    Wpe8 = jnp.zeros((PW, 8), f32).at[:5, :H].set(W_pe)
    bpe8 = jnp.zeros((1, 8), f32).at[0, :H].set(b_pe)
    lg = jnp.logical_not(global_attn).astype(f32).reshape(1, 1)
    mask2 = cluster_mask.reshape(BN, M)

    TB2 = 128
    g2 = BN // TB2
    out = pl.pallas_call(
        functools.partial(_attn_body, tb=TB2, m=M),
        grid=(g2,),
        in_specs=[
            pl.BlockSpec((TB2, C), lambda i: (i, 0)),
            pl.BlockSpec((TB2 * M, CP), lambda i: (i, 0)),
            pl.BlockSpec((TB2 * M, PW), lambda i: (i, 0)),
            pl.BlockSpec((TB2, M), lambda i: (i, 0)),
            pl.BlockSpec((1, 1), lambda i: (0, 0), memory_space=pltpu.SMEM),
            pl.BlockSpec((C, 8), lambda i: (0, 0)),
            pl.BlockSpec((8, C), lambda i: (0, 0)),
            pl.BlockSpec((PW, 8), lambda i: (0, 0)),
            pl.BlockSpec((1, 8), lambda i: (0, 0)),
            pl.BlockSpec((1, C), lambda i: (0, 0)),
            pl.BlockSpec((1, C), lambda i: (0, 0)),
            pl.BlockSpec((C, C), lambda i: (0, 0)),
            pl.BlockSpec((1, C), lambda i: (0, 0)),
        ],
        out_specs=pl.BlockSpec((TB2, C), lambda i: (i, 0)),
        out_shape=jax.ShapeDtypeStruct((BN, C), f32),
    )(q2, kvg, peg, mask2, lg, S, Srep, Wpe8, bpe8,
      blank_k.reshape(1, C), blank_v.reshape(1, C), W_proj,
      b_proj.reshape(1, C))

    return out.reshape(B, N, C)


# speedup vs baseline: 2.6957x; 1.0001x over previous
import functools

import jax
import jax.numpy as jnp
from jax import lax
from jax.experimental import pallas as pl
from jax.experimental.pallas import tpu as pltpu
from jax.experimental.pallas import tpu_sc as plsc

_NC = 2
_NS = 16
_NW = _NC * _NS


def _proj_body(x_ref, wq_ref, bq_ref, wk_ref, bk_ref, wv_ref, bv_ref,
               q_ref, kvp_ref):
    x = x_ref[...]
    tb = x.shape[0]
    q_ref[...] = jnp.dot(x, wq_ref[...]) + bq_ref[...]
    k = jnp.dot(x, wk_ref[...]) + bk_ref[...]
    v = jnp.dot(x, wv_ref[...]) + bv_ref[...]
    ku = lax.bitcast_convert_type(k.astype(jnp.bfloat16),
                                  jnp.uint16).astype(jnp.uint32)
    vu = lax.bitcast_convert_type(v.astype(jnp.bfloat16),
                                  jnp.uint16).astype(jnp.uint32)
    w = lax.bitcast_convert_type((ku << 16) | vu, jnp.float32)
    kvp_ref[...] = jnp.concatenate(
        [w, jnp.zeros((tb, 64), jnp.float32)], axis=1)


def _attn_body(q_ref, kvg_ref, peg_ref, mask_ref, lg_ref, s_ref, srep_ref,
               wpe_ref, bpe_ref, blankk_ref, blankv_ref, wproj_ref, bproj_ref,
               out_ref, *, tb, m):
    c = q_ref.shape[-1]
    q = q_ref[...]
    s_sel = s_ref[...]
    wu = lax.bitcast_convert_type(kvg_ref[...][:, :c], jnp.uint32)
    kg = lax.bitcast_convert_type(wu & jnp.uint32(0xFFFF0000), jnp.float32)
    vg = lax.bitcast_convert_type(wu << 16, jnp.float32)
    qe = jnp.broadcast_to(q[:, None, :], (tb, m, c)).reshape(tb * m, c)
    scores = jnp.dot(qe * kg, s_sel)
    pe = jnp.dot(peg_ref[...], wpe_ref[...]) + bpe_ref[...]
    lg = lg_ref[0, 0]
    s3 = scores.reshape(tb, m, 8) + pe.reshape(tb, m, 8)
    s3 = s3 + ((1.0 - mask_ref[...]) * (-100.0) * lg)[:, :, None]
    bl = jnp.clip(jnp.dot(q * blankk_ref[...], s_sel), -5.0, 5.0)
    mx = jnp.maximum(jnp.max(s3, axis=1), bl)
    e3 = jnp.exp(s3 - mx[:, None, :])
    eb = jnp.exp(bl - mx)
    den = jnp.sum(e3, axis=1) + eb
    attn = (e3 / den[:, None, :]).reshape(tb * m, 8)
    ar = jnp.dot(attn, srep_ref[...])
    out = jnp.sum((ar * vg).reshape(tb, m, c), axis=1)
    out = out + jnp.dot(eb / den, srep_ref[...]) * blankv_ref[...]
    out_ref[...] = jnp.dot(out, wproj_ref[...]) + bproj_ref[...]


def _make_sc_gather(rows, width, chk, per_w, nbuf, tc_tiling):
    n_chunks = per_w // chk
    t_steps = n_chunks // nbuf
    mesh = plsc.VectorSubcoreMesh(core_axis_name="c", subcore_axis_name="s")

    @functools.partial(
        pl.kernel,
        mesh=mesh,
        out_type=jax.ShapeDtypeStruct((rows, width), jnp.float32),
        scratch_types=(
            [pltpu.VMEM((per_w,), jnp.int32),
             pltpu.VMEM((nbuf * chk, width), jnp.float32)]
            + [pltpu.SemaphoreType.DMA] * (2 * nbuf)
        ),
        compiler_params=pltpu.CompilerParams(use_tc_tiling_on_sc=tc_tiling),
    )
    def sc_gather(idx_hbm, table_hbm, out_hbm, idx_all, bufs, *sems):
        semg = sems[:nbuf]
        semw = sems[nbuf:]
        wid = lax.axis_index("s") * _NC + lax.axis_index("c")
        w0 = wid * per_w
        pltpu.sync_copy(idx_hbm.at[pl.ds(w0, per_w)], idx_all)

        def gat(i, b):
            return pltpu.make_async_copy(
                table_hbm.at[idx_all.at[pl.ds(i * chk, chk)]],
                bufs.at[pl.ds(b * chk, chk)], semg[b])

        def sto(i, b):
            return pltpu.make_async_copy(
                bufs.at[pl.ds(b * chk, chk)],
                out_hbm.at[pl.ds(w0 + i * chk, chk)], semw[b])

        for b in range(nbuf):
            gat(b, b).start()

        def body(t, carry):
            for b in range(nbuf):
                i = t * nbuf + b
                gat(i, b).wait()
                sto(i, b).start()

                @pl.when(t < t_steps - 1)
                def _():
                    sto(i, b).wait()
                    gat(i + nbuf, b).start()

            return carry

        lax.fori_loop(0, t_steps, body, 0)
        for b in range(nbuf):
            sto((t_steps - 1) * nbuf + b, b).wait()

    return sc_gather


def kernel(feat, member_idx, cluster_mask, pe_idx, global_attn, pre_table,
           W_q, b_q, W_kv, b_kv, blank_k, blank_v, W_pe, b_pe, W_proj, b_proj):
    B, N, C = feat.shape
    M = member_idx.shape[-1]
    H = W_pe.shape[1]
    CH = C // H
    T = pre_table.shape[0]
    BN = B * N
    R = BN * M
    CP = 256
    scale = jnp.float32(CH) ** -0.5

    f32 = jnp.float32
    x = feat.reshape(BN, C)

    hcol = jnp.arange(C)
    kcols = (hcol // CH) * (2 * CH) + (hcol % CH)
    vcols = kcols + CH
    Wk = W_kv[:, kcols]
    Wv = W_kv[:, vcols]
    bk = b_kv[kcols].reshape(1, C)
    bv = b_kv[vcols].reshape(1, C)

    TB1 = 256
    g1 = BN // TB1
    q2, kvp = pl.pallas_call(
        _proj_body,
        grid=(g1,),
        in_specs=[
            pl.BlockSpec((TB1, C), lambda i: (i, 0)),
            pl.BlockSpec((C, C), lambda i: (0, 0)),
            pl.BlockSpec((1, C), lambda i: (0, 0)),
            pl.BlockSpec((C, C), lambda i: (0, 0)),
            pl.BlockSpec((1, C), lambda i: (0, 0)),
            pl.BlockSpec((C, C), lambda i: (0, 0)),
            pl.BlockSpec((1, C), lambda i: (0, 0)),
        ],
        out_specs=[
            pl.BlockSpec((TB1, C), lambda i: (i, 0)),
            pl.BlockSpec((TB1, CP), lambda i: (i, 0)),
        ],
        out_shape=[jax.ShapeDtypeStruct((BN, C), f32),
                   jax.ShapeDtypeStruct((BN, CP), f32)],
    )(x, W_q * scale, (b_q * scale).reshape(1, C), Wk, bk, Wv, bv)

    gidx = (member_idx.astype(jnp.int32)
            + (jnp.arange(B, dtype=jnp.int32) * N)[:, None, None]).reshape(R)
    pidx = pe_idx.astype(jnp.int32).reshape(R)
    PW = 8
    pre8 = jnp.zeros((T, PW), f32).at[:, :5].set(pre_table)

    per_w = R // _NW
    kvg = _make_sc_gather(R, CP, 64, per_w, 6, True)(gidx, kvp)
    peg = _make_sc_gather(R, PW, 128, per_w, 3, False)(pidx, pre8)

    S = ((hcol[:, None] // CH) == jnp.arange(8)[None, :]).astype(f32)
    Srep = S.T
    Wpe8 = jnp.zeros((PW, 8), f32).at[:5, :H].set(W_pe)
    bpe8 = jnp.zeros((1, 8), f32).at[0, :H].set(b_pe)
    lg = jnp.logical_not(global_attn).astype(f32).reshape(1, 1)
    mask2 = cluster_mask.reshape(BN, M)

    TB2 = 128
    g2 = BN // TB2
    out = pl.pallas_call(
        functools.partial(_attn_body, tb=TB2, m=M),
        grid=(g2,),
        in_specs=[
            pl.BlockSpec((TB2, C), lambda i: (i, 0)),
            pl.BlockSpec((TB2 * M, CP), lambda i: (i, 0)),
            pl.BlockSpec((TB2 * M, PW), lambda i: (i, 0)),
            pl.BlockSpec((TB2, M), lambda i: (i, 0)),
            pl.BlockSpec((1, 1), lambda i: (0, 0), memory_space=pltpu.SMEM),
            pl.BlockSpec((C, 8), lambda i: (0, 0)),
            pl.BlockSpec((8, C), lambda i: (0, 0)),
            pl.BlockSpec((PW, 8), lambda i: (0, 0)),
            pl.BlockSpec((1, 8), lambda i: (0, 0)),
            pl.BlockSpec((1, C), lambda i: (0, 0)),
            pl.BlockSpec((1, C), lambda i: (0, 0)),
            pl.BlockSpec((C, C), lambda i: (0, 0)),
            pl.BlockSpec((1, C), lambda i: (0, 0)),
        ],
        out_specs=pl.BlockSpec((TB2, C), lambda i: (i, 0)),
        out_shape=jax.ShapeDtypeStruct((BN, C), f32),
    )(q2, kvg, peg, mask2, lg, S, Srep, Wpe8, bpe8,
      blank_k.reshape(1, C), blank_v.reshape(1, C), W_proj,
      b_proj.reshape(1, C))

    return out.reshape(B, N, C)


# speedup vs baseline: 2.6960x; 1.0001x over previous
import functools

import jax
import jax.numpy as jnp
from jax import lax
from jax.experimental import pallas as pl
from jax.experimental.pallas import tpu as pltpu
from jax.experimental.pallas import tpu_sc as plsc

_NC = 2
_NS = 16
_NW = _NC * _NS


def _proj_body(x_ref, wq_ref, bq_ref, wk_ref, bk_ref, wv_ref, bv_ref,
               q_ref, kvp_ref):
    x = x_ref[...]
    tb = x.shape[0]
    q_ref[...] = jnp.dot(x, wq_ref[...]) + bq_ref[...]
    k = jnp.dot(x, wk_ref[...]) + bk_ref[...]
    v = jnp.dot(x, wv_ref[...]) + bv_ref[...]
    ku = lax.bitcast_convert_type(k.astype(jnp.bfloat16),
                                  jnp.uint16).astype(jnp.uint32)
    vu = lax.bitcast_convert_type(v.astype(jnp.bfloat16),
                                  jnp.uint16).astype(jnp.uint32)
    w = lax.bitcast_convert_type((ku << 16) | vu, jnp.float32)
    kvp_ref[...] = jnp.concatenate(
        [w, jnp.zeros((tb, 64), jnp.float32)], axis=1)


def _attn_body(q_ref, kvg_ref, peg_ref, mask_ref, lg_ref, s_ref, srep_ref,
               wpe_ref, bpe_ref, blankk_ref, blankv_ref, wproj_ref, bproj_ref,
               out_ref, *, tb, m):
    c = q_ref.shape[-1]
    q = q_ref[...]
    s_sel = s_ref[...]
    wu = lax.bitcast_convert_type(kvg_ref[...][:, :c], jnp.uint32)
    kg = lax.bitcast_convert_type(wu & jnp.uint32(0xFFFF0000), jnp.float32)
    vg = lax.bitcast_convert_type(wu << 16, jnp.float32)
    qe = jnp.broadcast_to(q[:, None, :], (tb, m, c)).reshape(tb * m, c)
    scores = jnp.dot(qe * kg, s_sel)
    pe = jnp.dot(peg_ref[...], wpe_ref[...]) + bpe_ref[...]
    lg = lg_ref[0, 0]
    s3 = scores.reshape(tb, m, 8) + pe.reshape(tb, m, 8)
    s3 = s3 + ((1.0 - mask_ref[...]) * (-100.0) * lg)[:, :, None]
    bl = jnp.clip(jnp.dot(q * blankk_ref[...], s_sel), -5.0, 5.0)
    mx = jnp.maximum(jnp.max(s3, axis=1), bl)
    e3 = jnp.exp(s3 - mx[:, None, :])
    eb = jnp.exp(bl - mx)
    den = jnp.sum(e3, axis=1) + eb
    attn = (e3 / den[:, None, :]).reshape(tb * m, 8)
    ar = jnp.dot(attn, srep_ref[...])
    out = jnp.sum((ar * vg).reshape(tb, m, c), axis=1)
    out = out + jnp.dot(eb / den, srep_ref[...]) * blankv_ref[...]
    out_ref[...] = jnp.dot(out, wproj_ref[...]) + bproj_ref[...]


def _make_sc_gather(rows, width, chk, per_w, nbuf, tc_tiling):
    n_chunks = per_w // chk
    t_steps = n_chunks // nbuf
    mesh = plsc.VectorSubcoreMesh(core_axis_name="c", subcore_axis_name="s")

    @functools.partial(
        pl.kernel,
        mesh=mesh,
        out_type=jax.ShapeDtypeStruct((rows, width), jnp.float32),
        scratch_types=(
            [pltpu.VMEM((per_w,), jnp.int32),
             pltpu.VMEM((nbuf * chk, width), jnp.float32)]
            + [pltpu.SemaphoreType.DMA] * (2 * nbuf)
        ),
        compiler_params=pltpu.CompilerParams(use_tc_tiling_on_sc=tc_tiling),
    )
    def sc_gather(idx_hbm, table_hbm, out_hbm, idx_all, bufs, *sems):
        semg = sems[:nbuf]
        semw = sems[nbuf:]
        wid = lax.axis_index("s") * _NC + lax.axis_index("c")
        w0 = wid * per_w
        pltpu.sync_copy(idx_hbm.at[pl.ds(w0, per_w)], idx_all)

        def gat(i, b):
            return pltpu.make_async_copy(
                table_hbm.at[idx_all.at[pl.ds(i * chk, chk)]],
                bufs.at[pl.ds(b * chk, chk)], semg[b])

        def sto(i, b):
            return pltpu.make_async_copy(
                bufs.at[pl.ds(b * chk, chk)],
                out_hbm.at[pl.ds(w0 + i * chk, chk)], semw[b])

        for b in range(nbuf):
            gat(b, b).start()

        def body(t, carry):
            for b in range(nbuf):
                i = t * nbuf + b
                gat(i, b).wait()
                sto(i, b).start()

                @pl.when(t < t_steps - 1)
                def _():
                    sto(i, b).wait()
                    gat(i + nbuf, b).start()

            return carry

        lax.fori_loop(0, t_steps, body, 0)
        for b in range(nbuf):
            sto((t_steps - 1) * nbuf + b, b).wait()

    return sc_gather


def kernel(feat, member_idx, cluster_mask, pe_idx, global_attn, pre_table,
           W_q, b_q, W_kv, b_kv, blank_k, blank_v, W_pe, b_pe, W_proj, b_proj):
    B, N, C = feat.shape
    M = member_idx.shape[-1]
    H = W_pe.shape[1]
    CH = C // H
    T = pre_table.shape[0]
    BN = B * N
    R = BN * M
    CP = 256
    scale = jnp.float32(CH) ** -0.5

    f32 = jnp.float32
    x = feat.reshape(BN, C)

    hcol = jnp.arange(C)
    kcols = (hcol // CH) * (2 * CH) + (hcol % CH)
    vcols = kcols + CH
    Wk = W_kv[:, kcols]
    Wv = W_kv[:, vcols]
    bk = b_kv[kcols].reshape(1, C)
    bv = b_kv[vcols].reshape(1, C)

    TB1 = 256
    g1 = BN // TB1
    q2, kvp = pl.pallas_call(
        _proj_body,
        grid=(g1,),
        in_specs=[
            pl.BlockSpec((TB1, C), lambda i: (i, 0)),
            pl.BlockSpec((C, C), lambda i: (0, 0)),
            pl.BlockSpec((1, C), lambda i: (0, 0)),
            pl.BlockSpec((C, C), lambda i: (0, 0)),
            pl.BlockSpec((1, C), lambda i: (0, 0)),
            pl.BlockSpec((C, C), lambda i: (0, 0)),
            pl.BlockSpec((1, C), lambda i: (0, 0)),
        ],
        out_specs=[
            pl.BlockSpec((TB1, C), lambda i: (i, 0)),
            pl.BlockSpec((TB1, CP), lambda i: (i, 0)),
        ],
        out_shape=[jax.ShapeDtypeStruct((BN, C), f32),
                   jax.ShapeDtypeStruct((BN, CP), f32)],
    )(x, W_q * scale, (b_q * scale).reshape(1, C), Wk, bk, Wv, bv)

    gidx = (member_idx.astype(jnp.int32)
            + (jnp.arange(B, dtype=jnp.int32) * N)[:, None, None]).reshape(R)
    pidx = pe_idx.astype(jnp.int32).reshape(R)
    PW = 8
    pre8 = jnp.zeros((T, PW), f32).at[:, :5].set(pre_table)

    per_w = R // _NW
    kvg = _make_sc_gather(R, CP, 128, per_w, 3, True)(gidx, kvp)
    peg = _make_sc_gather(R, PW, 128, per_w, 3, False)(pidx, pre8)

    S = ((hcol[:, None] // CH) == jnp.arange(8)[None, :]).astype(f32)
    Srep = S.T
    Wpe8 = jnp.zeros((PW, 8), f32).at[:5, :H].set(W_pe)
    bpe8 = jnp.zeros((1, 8), f32).at[0, :H].set(b_pe)
    lg = jnp.logical_not(global_attn).astype(f32).reshape(1, 1)
    mask2 = cluster_mask.reshape(BN, M)

    TB2 = 128
    g2 = BN // TB2
    out = pl.pallas_call(
        functools.partial(_attn_body, tb=TB2, m=M),
        grid=(g2,),
        in_specs=[
            pl.BlockSpec((TB2, C), lambda i: (i, 0)),
            pl.BlockSpec((TB2 * M, CP), lambda i: (i, 0)),
            pl.BlockSpec((TB2 * M, PW), lambda i: (i, 0)),
            pl.BlockSpec((TB2, M), lambda i: (i, 0)),
            pl.BlockSpec((1, 1), lambda i: (0, 0), memory_space=pltpu.SMEM),
            pl.BlockSpec((C, 8), lambda i: (0, 0)),
            pl.BlockSpec((8, C), lambda i: (0, 0)),
            pl.BlockSpec((PW, 8), lambda i: (0, 0)),
            pl.BlockSpec((1, 8), lambda i: (0, 0)),
            pl.BlockSpec((1, C), lambda i: (0, 0)),
            pl.BlockSpec((1, C), lambda i: (0, 0)),
            pl.BlockSpec((C, C), lambda i: (0, 0)),
            pl.BlockSpec((1, C), lambda i: (0, 0)),
        ],
        out_specs=pl.BlockSpec((TB2, C), lambda i: (i, 0)),
        out_shape=jax.ShapeDtypeStruct((BN, C), f32),
    )(q2, kvg, peg, mask2, lg, S, Srep, Wpe8, bpe8,
      blank_k.reshape(1, C), blank_v.reshape(1, C), W_proj,
      b_proj.reshape(1, C))

    return out.reshape(B, N, C)
